# baseline (speedup 1.0000x reference)
import functools

import jax
import jax.numpy as jnp
from jax import lax
from jax.experimental import pallas as pl
from jax.experimental.pallas import tpu as pltpu
from jax.experimental.pallas import tpu_sc as plsc

NUM_FIELDS = 26
FIELD_SIZE = 100000
EMBED = 16
LANES = 16
NUM_WORKERS = 32
CHUNK = 4 * NUM_FIELDS
HALF = 2 * CHUNK
HBROWS = HALF // NUM_FIELDS


def _make_kernel(batch: int, n_rows: int):
    per_w = n_rows // NUM_WORKERS
    n_halves = per_w // HALF
    pairs = n_halves // 2
    b_per_w = batch // NUM_WORKERS
    mesh = plsc.VectorSubcoreMesh(core_axis_name="c", subcore_axis_name="s")

    @functools.partial(
        pl.kernel,
        out_type=jax.ShapeDtypeStruct((batch, NUM_FIELDS, EMBED), jnp.float32),
        mesh=mesh,
        compiler_params=pltpu.CompilerParams(needs_layout_passes=False),
        scratch_types=[
            pltpu.VMEM((per_w,), jnp.int32),
            pltpu.VMEM((HALF,), jnp.int32),
            pltpu.VMEM((HALF,), jnp.int32),
            pltpu.VMEM((HALF, 128), jnp.float32),
            pltpu.VMEM((HALF, 128), jnp.float32),
            pltpu.VMEM((HALF, EMBED), jnp.float32),
            pltpu.VMEM((HALF, EMBED), jnp.float32),
            pltpu.SemaphoreType.DMA,
            pltpu.SemaphoreType.DMA,
            pltpu.SemaphoreType.DMA,
            pltpu.SemaphoreType.DMA,
        ],
    )
    def run(x_hbm, tbl_hbm, out_hbm, idx, ga, gb, tiles_a, tiles_b,
            rows_a, rows_b, gsem_a, gsem_b, ssem_a, ssem_b):
        wid = lax.axis_index("s") * 2 + lax.axis_index("c")
        base = wid * per_w
        brow0 = wid * b_per_w
        pltpu.sync_copy(x_hbm.at[pl.ds(base, per_w)], idx)

        lane = lax.broadcasted_iota(jnp.int32, (LANES,), 0)

        def prep(h, gbuf):
            for v in range(HALF // LANES):
                off = pl.multiple_of(h * HALF + v * LANES, LANES)
                loc = pl.multiple_of(v * LANES, LANES)
                field = lax.rem(base + off + lane, NUM_FIELDS)
                r = idx[pl.ds(off, LANES)] + field * FIELD_SIZE
                idx[pl.ds(off, LANES)] = r
                gbuf[pl.ds(loc, LANES)] = lax.shift_right_logical(r, 3)

        def fire_gathers(gbuf, tiles, sem):
            for b in range(2):
                off = pl.multiple_of(b * CHUNK, 8)
                pltpu.async_copy(
                    tbl_hbm.at[gbuf.at[pl.ds(off, CHUNK)]],
                    tiles.at[pl.ds(b * CHUNK, CHUNK)], sem
                )

        def extract(h, tiles, rows):
            def blk(v, _):
                off = pl.multiple_of(h * HALF + v * LANES, LANES)
                jv = lax.bitwise_and(idx[pl.ds(off, LANES)], 7)
                rowv = lane + v * LANES
                col0 = jv * EMBED
                for c in range(EMBED):
                    cvec = lane * 0 + c
                    vals = plsc.load_gather(tiles, [rowv, col0 + cvec])
                    plsc.store_scatter(rows, [rowv, cvec], vals)
                return 0

            lax.fori_loop(0, HALF // LANES, blk, 0)

        def fire_stores(h, rows, sem):
            row = pl.multiple_of(brow0 + h * HBROWS, HBROWS)
            for r in range(HBROWS):
                pltpu.async_copy(
                    rows.at[pl.ds(r * NUM_FIELDS, NUM_FIELDS)],
                    out_hbm.at[row + r], sem
                )

        def drain_g(sem):
            for _ in range(2):
                pltpu.make_async_copy(
                    tbl_hbm.at[ga.at[pl.ds(0, CHUNK)]],
                    tiles_a.at[pl.ds(0, CHUNK)], sem
                ).wait()

        def drain_s(sem):
            for _ in range(HBROWS):
                pltpu.make_async_copy(
                    rows_a.at[pl.ds(0, NUM_FIELDS)], out_hbm.at[brow0], sem
                ).wait()

        prep(0, ga)
        fire_gathers(ga, tiles_a, gsem_a)
        prep(1, gb)
        fire_gathers(gb, tiles_b, gsem_b)
        drain_g(gsem_a)
        extract(0, tiles_a, rows_a)
        fire_stores(0, rows_a, ssem_a)

        def body(t, _):
            h0 = pl.multiple_of(2 * t, 2)
            h1 = h0 + 1
            prep(h0, ga)
            fire_gathers(ga, tiles_a, gsem_a)
            drain_g(gsem_b)
            extract(h1 - 2, tiles_b, rows_b)
            fire_stores(h1 - 2, rows_b, ssem_b)
            prep(h1, gb)
            fire_gathers(gb, tiles_b, gsem_b)
            drain_g(gsem_a)
            drain_s(ssem_a)
            extract(h0, tiles_a, rows_a)
            fire_stores(h0, rows_a, ssem_a)
            drain_s(ssem_b)
            return 0

        lax.fori_loop(1, pairs, body, 0)

        drain_g(gsem_b)
        extract(n_halves - 1, tiles_b, rows_b)
        fire_stores(n_halves - 1, rows_b, ssem_b)
        drain_s(ssem_a)
        drain_s(ssem_b)

    return run


def kernel(x, table):
    batch, num_fields = x.shape
    n_rows = batch * num_fields
    x_flat = x.reshape(n_rows)
    tblw = table.reshape(table.shape[0] // 8, 128)
    return _make_kernel(batch, n_rows)(x_flat, tblw)


# speedup vs baseline: 1.0540x; 1.0540x over previous
import functools

import jax
import jax.numpy as jnp
from jax import lax
from jax.experimental import pallas as pl
from jax.experimental.pallas import tpu as pltpu
from jax.experimental.pallas import tpu_sc as plsc

NUM_FIELDS = 26
FIELD_SIZE = 100000
EMBED = 16
LANES = 16
NUM_WORKERS = 32
BROWS = 4
CHUNK = BROWS * NUM_FIELDS
K = 4
GSZ = K * CHUNK


def _make_kernel(batch: int, n_rows: int):
    per_w = n_rows // NUM_WORKERS
    n_groups = per_w // GSZ
    pairs = n_groups // 2
    b_per_w = batch // NUM_WORKERS
    mesh = plsc.VectorSubcoreMesh(core_axis_name="c", subcore_axis_name="s")

    @functools.partial(
        pl.kernel,
        out_type=jax.ShapeDtypeStruct((batch, NUM_FIELDS, EMBED), jnp.float32),
        mesh=mesh,
        compiler_params=pltpu.CompilerParams(
            use_tc_tiling_on_sc=False, needs_layout_passes=False),
        scratch_types=[
            pltpu.VMEM((per_w,), jnp.int32),
            pltpu.VMEM((K, CHUNK, EMBED), jnp.float32),
            pltpu.VMEM((K, CHUNK, EMBED), jnp.float32),
            pltpu.SemaphoreType.DMA,
            pltpu.SemaphoreType.DMA,
            pltpu.SemaphoreType.DMA,
            pltpu.SemaphoreType.DMA,
        ],
    )
    def run(x_hbm, table_hbm, out_hbm, idx_v, buf_a, buf_b,
            gsem_a, gsem_b, ssem_a, ssem_b):
        wid = lax.axis_index("s") * 2 + lax.axis_index("c")
        base = wid * per_w
        brow0 = wid * b_per_w
        pltpu.sync_copy(x_hbm.at[pl.ds(base, per_w)], idx_v)

        lane = lax.broadcasted_iota(jnp.int32, (LANES,), 0)

        def prep(g):
            for v in range(GSZ // LANES):
                off = pl.multiple_of(g * GSZ + v * LANES, LANES)
                field = lax.rem(base + off + lane, NUM_FIELDS)
                idx_v[pl.ds(off, LANES)] = (
                    idx_v[pl.ds(off, LANES)] + field * FIELD_SIZE
                )

        def fire_gathers(g, buf, sem):
            for b in range(K):
                off = pl.multiple_of(g * GSZ + b * CHUNK, 8)
                pltpu.async_copy(
                    table_hbm.at[idx_v.at[pl.ds(off, CHUNK)]], buf.at[b], sem
                )

        def fire_stores(g, buf, sem):
            for b in range(K):
                row = pl.multiple_of(brow0 + g * (K * BROWS) + b * BROWS, BROWS)
                for r in range(BROWS):
                    pltpu.async_copy(
                        buf.at[b, pl.ds(r * NUM_FIELDS, NUM_FIELDS)],
                        out_hbm.at[row + r], sem
                    )

        def drain_g(sem, n):
            for _ in range(n):
                pltpu.make_async_copy(
                    table_hbm.at[idx_v.at[pl.ds(0, CHUNK)]], buf_a.at[0], sem
                ).wait()

        def drain_s(sem, n):
            for _ in range(n * BROWS):
                pltpu.make_async_copy(
                    buf_a.at[0, pl.ds(0, NUM_FIELDS)], out_hbm.at[brow0], sem
                ).wait()

        prep(0)
        fire_gathers(0, buf_a, gsem_a)
        prep(1)
        fire_gathers(1, buf_b, gsem_b)
        drain_g(gsem_a, K)
        fire_stores(0, buf_a, ssem_a)

        def body(t, _):
            g0 = pl.multiple_of(2 * t, 2)
            g1 = g0 + 1
            prep(g0)
            drain_s(ssem_a, K)
            fire_gathers(g0, buf_a, gsem_a)
            drain_g(gsem_b, K)
            fire_stores(g1 - 2, buf_b, ssem_b)
            prep(g1)
            drain_s(ssem_b, K)
            fire_gathers(g1, buf_b, gsem_b)
            drain_g(gsem_a, K)
            fire_stores(g0, buf_a, ssem_a)
            return 0

        lax.fori_loop(1, pairs, body, 0)

        drain_g(gsem_b, K)
        fire_stores(n_groups - 1, buf_b, ssem_b)
        drain_s(ssem_a, K)
        drain_s(ssem_b, K)

    return run


def kernel(x, table):
    batch, num_fields = x.shape
    n_rows = batch * num_fields
    x_flat = x.reshape(n_rows)
    return _make_kernel(batch, n_rows)(x_flat, table)
